# Initial kernel scaffold; baseline (speedup 1.0000x reference)
import jax
import jax.numpy as jnp
from jax.experimental import pallas as pl


def kernel(x, task_keys, temperature, W1, b1, W2, b2):
    raise NotImplementedError("write your pallas kernel here")


# speedup vs baseline: 1.7449x; 1.7449x over previous
import functools

import jax
import jax.numpy as jnp
from jax.experimental import pallas as pl
from jax.experimental.pallas import tpu as pltpu

N_TASKS = 10
D_MODEL = 4096
HIDDEN = 128
CLASSES = 3
BATCH = 1024

_ROUTE_BLOCK = 256
_CLS_BLOCK = 256


def _route_kernel(x_ref, keys_ref, temp_ref, logits_ref, probs_ref, task_ref,
                  counts_scr):
    i = pl.program_id(0)
    nb = pl.num_programs(0)

    xb = x_ref[...]
    keys = keys_ref[...]
    ssk = jnp.sum(keys * keys, axis=-1, keepdims=True)
    keysn = keys * jax.lax.rsqrt(jnp.maximum(ssk, 1e-12))
    ssx = jnp.sum(xb * xb, axis=-1, keepdims=True)
    xinv = jax.lax.rsqrt(jnp.maximum(ssx, 1e-12))
    sim = jnp.dot(xb, keysn.T, preferred_element_type=jnp.float32) * xinv
    logits = sim / temp_ref[0]
    logits_ref[...] = logits

    m = jnp.max(logits, axis=-1, keepdims=True)
    e = jnp.exp(logits - m)
    probs_ref[...] = e / jnp.sum(e, axis=-1, keepdims=True)

    col = jax.lax.broadcasted_iota(jnp.int32, logits.shape, 1)
    pred = jnp.min(jnp.where(logits == m, col, N_TASKS), axis=-1,
                   keepdims=True)

    onehot = (pred == jax.lax.broadcasted_iota(
        jnp.int32, (logits.shape[0], N_TASKS), 1)).astype(jnp.int32)
    c = jnp.sum(onehot, axis=0, keepdims=True)

    @pl.when(i == 0)
    def _():
        counts_scr[...] = jnp.zeros_like(counts_scr)

    counts_scr[...] += c

    @pl.when(i == nb - 1)
    def _():
        counts = counts_scr[...]
        mc = jnp.max(counts)
        tcol = jax.lax.broadcasted_iota(jnp.int32, counts.shape, 1)
        task_ref[0, 0] = jnp.min(jnp.where(counts == mc, tcol, N_TASKS))


def _cls_kernel(task_ref, x_ref, w1_ref, b1_ref, w2_ref, b2_ref, out_ref):
    del task_ref
    xb = x_ref[...]
    w1 = w1_ref[0]
    h = jnp.dot(xb, w1, preferred_element_type=jnp.float32) + b1_ref[0]
    h = jnp.maximum(h, 0.0)
    w2 = w2_ref[0]
    out_ref[...] = jnp.dot(h, w2, preferred_element_type=jnp.float32) \
        + b2_ref[0]


@functools.partial(jax.jit)
def kernel(x, task_keys, temperature, W1, b1, W2, b2):
    nb = BATCH // _ROUTE_BLOCK
    task_logits, task_probs, ptask = pl.pallas_call(
        _route_kernel,
        grid=(nb,),
        in_specs=[
            pl.BlockSpec((_ROUTE_BLOCK, D_MODEL), lambda i: (i, 0)),
            pl.BlockSpec((N_TASKS, D_MODEL), lambda i: (0, 0)),
            pl.BlockSpec(memory_space=pltpu.SMEM),
        ],
        out_specs=[
            pl.BlockSpec((_ROUTE_BLOCK, N_TASKS), lambda i: (i, 0)),
            pl.BlockSpec((_ROUTE_BLOCK, N_TASKS), lambda i: (i, 0)),
            pl.BlockSpec(memory_space=pltpu.SMEM),
        ],
        out_shape=[
            jax.ShapeDtypeStruct((BATCH, N_TASKS), jnp.float32),
            jax.ShapeDtypeStruct((BATCH, N_TASKS), jnp.float32),
            jax.ShapeDtypeStruct((1, 1), jnp.int32),
        ],
        scratch_shapes=[pltpu.VMEM((1, N_TASKS), jnp.int32)],
    )(x, task_keys, temperature)

    nb2 = BATCH // _CLS_BLOCK
    logits = pl.pallas_call(
        _cls_kernel,
        grid_spec=pltpu.PrefetchScalarGridSpec(
            num_scalar_prefetch=1,
            grid=(nb2,),
            in_specs=[
                pl.BlockSpec((_CLS_BLOCK, D_MODEL), lambda i, t: (i, 0)),
                pl.BlockSpec((1, D_MODEL, HIDDEN), lambda i, t: (t[0], 0, 0)),
                pl.BlockSpec((1, 1, HIDDEN), lambda i, t: (t[0], 0, 0)),
                pl.BlockSpec((1, HIDDEN, CLASSES), lambda i, t: (t[0], 0, 0)),
                pl.BlockSpec((1, 1, CLASSES), lambda i, t: (t[0], 0, 0)),
            ],
            out_specs=pl.BlockSpec((_CLS_BLOCK, CLASSES), lambda i, t: (i, 0)),
        ),
        out_shape=jax.ShapeDtypeStruct((BATCH, CLASSES), jnp.float32),
    )(ptask.reshape((1,)), x, W1,
      b1.reshape(N_TASKS, 1, HIDDEN), W2, b2.reshape(N_TASKS, 1, CLASSES))

    return (logits, task_logits, task_probs)


# speedup vs baseline: 2.0186x; 1.1568x over previous
import functools

import jax
import jax.numpy as jnp
from jax.experimental import pallas as pl
from jax.experimental.pallas import tpu as pltpu

N_TASKS = 10
D_MODEL = 4096
HIDDEN = 128
CLASSES = 3
BATCH = 1024

_BLOCK = 256


def _fused_kernel(x_ref, keys_ref, temp_ref, w1_hbm, b1_ref, w2_ref, b2_ref,
                  tl_ref, tp_ref, logits_ref,
                  x_vmem, w1_vmem, counts_scr, sem):
    i = pl.program_id(0)
    nb = pl.num_programs(0)

    xb = x_ref[...]
    x_vmem[pl.ds(i * _BLOCK, _BLOCK), :] = xb

    keys = keys_ref[...]
    ssk = jnp.sum(keys * keys, axis=-1, keepdims=True)
    keysn = keys * jax.lax.rsqrt(jnp.maximum(ssk, 1e-12))
    ssx = jnp.sum(xb * xb, axis=-1, keepdims=True)
    xinv = jax.lax.rsqrt(jnp.maximum(ssx, 1e-12))
    sim = jnp.dot(xb, keysn.T, preferred_element_type=jnp.float32) * xinv
    logits = sim / temp_ref[0]
    tl_ref[...] = logits

    m = jnp.max(logits, axis=-1, keepdims=True)
    e = jnp.exp(logits - m)
    tp_ref[...] = e / jnp.sum(e, axis=-1, keepdims=True)

    col = jax.lax.broadcasted_iota(jnp.int32, logits.shape, 1)
    pred = jnp.min(jnp.where(logits == m, col, N_TASKS), axis=-1,
                   keepdims=True)
    onehot = (pred == jax.lax.broadcasted_iota(
        jnp.int32, (_BLOCK, N_TASKS), 1)).astype(jnp.int32)
    c = jnp.sum(onehot, axis=0, keepdims=True)

    @pl.when(i == 0)
    def _():
        counts_scr[...] = jnp.zeros_like(counts_scr)

    counts_scr[...] += c

    @pl.when(i == nb - 1)
    def _():
        counts = counts_scr[...]
        mc = jnp.max(counts)
        tcol = jax.lax.broadcasted_iota(jnp.int32, counts.shape, 1)
        t = jnp.min(jnp.where(counts == mc, tcol, N_TASKS))

        cp = pltpu.make_async_copy(w1_hbm.at[t], w1_vmem, sem)
        cp.start()

        trow = jax.lax.broadcasted_iota(jnp.int32, (N_TASKS, 1), 0)
        b1v = jnp.sum(jnp.where(trow == t, b1_ref[...], 0.0), axis=0,
                      keepdims=True)
        b2v = jnp.sum(jnp.where(trow == t, b2_ref[...], 0.0), axis=0,
                      keepdims=True)
        trow3 = jax.lax.broadcasted_iota(jnp.int32, (N_TASKS, 1, 1), 0)
        w2 = jnp.sum(jnp.where(trow3 == t, w2_ref[...], 0.0), axis=0)

        cp.wait()
        w1 = w1_vmem[...]
        h = jnp.dot(x_vmem[...], w1,
                    preferred_element_type=jnp.float32) + b1v
        h = jnp.maximum(h, 0.0)
        logits_ref[...] = jnp.dot(
            h, w2, preferred_element_type=jnp.float32) + b2v


@functools.partial(jax.jit)
def kernel(x, task_keys, temperature, W1, b1, W2, b2):
    nb = BATCH // _BLOCK
    task_logits, task_probs, logits = pl.pallas_call(
        _fused_kernel,
        grid=(nb,),
        in_specs=[
            pl.BlockSpec((_BLOCK, D_MODEL), lambda i: (i, 0)),
            pl.BlockSpec((N_TASKS, D_MODEL), lambda i: (0, 0)),
            pl.BlockSpec(memory_space=pltpu.SMEM),
            pl.BlockSpec(memory_space=pltpu.MemorySpace.HBM),
            pl.BlockSpec((N_TASKS, HIDDEN), lambda i: (0, 0)),
            pl.BlockSpec((N_TASKS, HIDDEN, CLASSES), lambda i: (0, 0, 0)),
            pl.BlockSpec((N_TASKS, CLASSES), lambda i: (0, 0)),
        ],
        out_specs=[
            pl.BlockSpec((_BLOCK, N_TASKS), lambda i: (i, 0)),
            pl.BlockSpec((_BLOCK, N_TASKS), lambda i: (i, 0)),
            pl.BlockSpec((BATCH, CLASSES), lambda i: (0, 0)),
        ],
        out_shape=[
            jax.ShapeDtypeStruct((BATCH, N_TASKS), jnp.float32),
            jax.ShapeDtypeStruct((BATCH, N_TASKS), jnp.float32),
            jax.ShapeDtypeStruct((BATCH, CLASSES), jnp.float32),
        ],
        scratch_shapes=[
            pltpu.VMEM((BATCH, D_MODEL), jnp.float32),
            pltpu.VMEM((D_MODEL, HIDDEN), jnp.float32),
            pltpu.VMEM((1, N_TASKS), jnp.int32),
            pltpu.SemaphoreType.DMA,
        ],
    )(x, task_keys, temperature, W1, b1, W2, b2)

    return (logits, task_logits, task_probs)
